# baseline (speedup 1.0000x reference)
import jax
import jax.numpy as jnp
from jax.experimental import pallas as pl
from jax.experimental.pallas import tpu as pltpu

B, SEQ, D = 32, 196, 768
POOL, LENGTH, TOPK = 100, 10, 5
CS = 4
NCK = B // CS
DEPTH = 3


def _fused_kernel(x_ref, x_key_ref, prompt_ref, prompt_key_ref,
                  out_hbm, rs_ref, mean_s, ob0, ob1, ob2, ssem):
    j = pl.program_id(0)
    obufs = (ob0, ob1, ob2)

    @pl.when(j == 0)
    def _prep():
        xk = x_key_ref[...]
        xk = xk / jnp.maximum(
            jnp.sqrt(jnp.sum(xk * xk, axis=1, keepdims=True)), 1e-12)
        pk = prompt_key_ref[...]
        pk = pk / jnp.maximum(
            jnp.sqrt(jnp.sum(pk * pk, axis=1, keepdims=True)), 1e-12)

        sim = jnp.dot(xk, pk.T, preferred_element_type=jnp.float32)
        m = jnp.max(sim, axis=1, keepdims=True)
        e = jnp.exp(sim - m)
        p = e / jnp.sum(e, axis=1, keepdims=True)

        for l in range(LENGTH):
            mean_s[:, l, :] = jnp.dot(
                p, prompt_ref[:, l, :],
                preferred_element_type=jnp.float32) * (1.0 / POOL)

        iota = jax.lax.broadcasted_iota(jnp.int32, (B, POOL), 1)
        v = sim
        total = jnp.float32(0.0)
        for _ in range(TOPK):
            mx = jnp.max(v, axis=1, keepdims=True)
            idx = jnp.min(jnp.where(v >= mx, iota, jnp.int32(POOL)),
                          axis=1, keepdims=True)
            total = total + jnp.sum(mx)
            v = jnp.where(iota == idx, -jnp.inf, v)
        rs_ref[...] = jnp.full((1, 1), total * (1.0 / B), jnp.float32)

    def store(c, slot):
        return pltpu.make_async_copy(
            obufs[slot],
            out_hbm.at[pl.ds(c * CS, CS), :, :],
            ssem.at[slot])

    @pl.when(j >= DEPTH)
    def _reclaim():
        for s in range(DEPTH):
            @pl.when(jax.lax.rem(j, DEPTH) == s)
            def _():
                store(j - DEPTH, s).wait()

    for s in range(DEPTH):
        @pl.when(jax.lax.rem(j, DEPTH) == s)
        def _assemble():
            ob = obufs[s]
            ob[:, LENGTH:, :] = x_ref[...]
            ob[:, :LENGTH, :] = mean_s[pl.ds(j * CS, CS), :, :]
            store(j, s).start()

    @pl.when(j == NCK - 1)
    def _drain():
        for c in range(NCK - DEPTH, NCK):
            store(c, c % DEPTH).wait()


@jax.jit
def kernel(x_embed, x_key, prompt, prompt_key):
    obuf_t = pltpu.VMEM((CS, LENGTH + SEQ, D), jnp.float32)
    out, rs = pl.pallas_call(
        _fused_kernel,
        grid=(NCK,),
        in_specs=[
            pl.BlockSpec((CS, SEQ, D), lambda j: (j, 0, 0)),
            pl.BlockSpec((B, 2 * D), lambda j: (0, 0)),
            pl.BlockSpec((POOL, LENGTH, D), lambda j: (0, 0, 0)),
            pl.BlockSpec((POOL, 2 * D), lambda j: (0, 0)),
        ],
        out_specs=[
            pl.BlockSpec(memory_space=pl.ANY),
            pl.BlockSpec((1, 1), lambda j: (0, 0)),
        ],
        out_shape=[
            jax.ShapeDtypeStruct((B, LENGTH + SEQ, D), jnp.float32),
            jax.ShapeDtypeStruct((1, 1), jnp.float32),
        ],
        scratch_shapes=[
            pltpu.VMEM((B, LENGTH, D), jnp.float32),
            obuf_t, obuf_t, obuf_t,
            pltpu.SemaphoreType.DMA((DEPTH,)),
        ],
    )(x_embed, x_key, prompt, prompt_key)
    return out, rs[0, 0]


# speedup vs baseline: 1.0754x; 1.0754x over previous
import jax
import jax.numpy as jnp
from jax.experimental import pallas as pl
from jax.experimental.pallas import tpu as pltpu

B, SEQ, D = 32, 196, 768
POOL, LENGTH, TOPK = 100, 10, 5
NC = 8
CS = B // NC


def _fused_kernel(x_hbm, x_key_ref, prompt_ref, prompt_key_ref,
                  out_hbm, rs_ref, xbuf, obuf, lsem, ssem):
    for c in range(NC):
        sl = slice(c * CS, (c + 1) * CS)
        pltpu.make_async_copy(
            x_hbm.at[sl, :, :], xbuf.at[sl, :, :], lsem.at[c]).start()

    xk = x_key_ref[...]
    xk = xk / jnp.maximum(
        jnp.sqrt(jnp.sum(xk * xk, axis=1, keepdims=True)), 1e-12)
    pk = prompt_key_ref[...]
    pk = pk / jnp.maximum(
        jnp.sqrt(jnp.sum(pk * pk, axis=1, keepdims=True)), 1e-12)

    sim = jnp.dot(xk, pk.T, preferred_element_type=jnp.float32)
    m = jnp.max(sim, axis=1, keepdims=True)
    e = jnp.exp(sim - m)
    p = e / jnp.sum(e, axis=1, keepdims=True)

    for l in range(LENGTH):
        obuf[:, l, :] = jnp.dot(
            p, prompt_ref[:, l, :],
            preferred_element_type=jnp.float32) * (1.0 / POOL)

    iota = jax.lax.broadcasted_iota(jnp.int32, (B, POOL), 1)
    v = sim
    total = jnp.float32(0.0)
    for _ in range(TOPK):
        mx = jnp.max(v, axis=1, keepdims=True)
        idx = jnp.min(jnp.where(v >= mx, iota, jnp.int32(POOL)),
                      axis=1, keepdims=True)
        total = total + jnp.sum(mx)
        v = jnp.where(iota == idx, -jnp.inf, v)
    rs_ref[...] = jnp.full((1, 1), total * (1.0 / B), jnp.float32)

    for c in range(NC):
        sl = slice(c * CS, (c + 1) * CS)
        pltpu.make_async_copy(
            x_hbm.at[sl, :, :], xbuf.at[sl, :, :], lsem.at[c]).wait()
        obuf[sl, LENGTH:, :] = xbuf[sl, :, :]
        pltpu.make_async_copy(
            obuf.at[sl, :, :], out_hbm.at[sl, :, :], ssem.at[c]).start()

    for c in range(NC):
        sl = slice(c * CS, (c + 1) * CS)
        pltpu.make_async_copy(
            obuf.at[sl, :, :], out_hbm.at[sl, :, :], ssem.at[c]).wait()


@jax.jit
def kernel(x_embed, x_key, prompt, prompt_key):
    out, rs = pl.pallas_call(
        _fused_kernel,
        in_specs=[
            pl.BlockSpec(memory_space=pl.ANY),
            pl.BlockSpec(memory_space=pltpu.MemorySpace.VMEM),
            pl.BlockSpec(memory_space=pltpu.MemorySpace.VMEM),
            pl.BlockSpec(memory_space=pltpu.MemorySpace.VMEM),
        ],
        out_specs=[
            pl.BlockSpec(memory_space=pl.ANY),
            pl.BlockSpec(memory_space=pltpu.MemorySpace.VMEM),
        ],
        out_shape=[
            jax.ShapeDtypeStruct((B, LENGTH + SEQ, D), jnp.float32),
            jax.ShapeDtypeStruct((1, 1), jnp.float32),
        ],
        scratch_shapes=[
            pltpu.VMEM((B, SEQ, D), jnp.float32),
            pltpu.VMEM((B, LENGTH + SEQ, D), jnp.float32),
            pltpu.SemaphoreType.DMA((NC,)),
            pltpu.SemaphoreType.DMA((NC,)),
        ],
    )(x_embed, x_key, prompt, prompt_key)
    return out, rs[0, 0]
